# baseline (speedup 1.0000x reference)
import functools

import jax
import jax.numpy as jnp
from jax import lax
from jax.experimental import pallas as pl
from jax.experimental.pallas import tpu as pltpu
from jax.experimental.pallas import tpu_sc as plsc

_N_NODES = 10000
_N_EDGES = 320000
_HID = 128
_EMB = 16

_NC = 2
_NS = 16
_NW = _NC * _NS
_EPT = _N_EDGES // _NW
_B = 80
_NCH = _EPT // _B
_DEPTH = 4
_NGRP = _NCH // _DEPTH


def _proj_body(x_ref, w1d_ref, w1s_ref, b1_ref, pd_ref, ps_ref):
    x = x_ref[...]
    pd_ref[...] = jnp.dot(x, w1d_ref[...], preferred_element_type=jnp.float32) + b1_ref[...]
    ps_ref[...] = jnp.dot(x, w1s_ref[...], preferred_element_type=jnp.float32)


def _proj(x, w1d, w1s, b1_row):
    return pl.pallas_call(
        _proj_body,
        out_shape=(
            jax.ShapeDtypeStruct((_N_NODES, _HID), jnp.float32),
            jax.ShapeDtypeStruct((_N_NODES, _HID), jnp.float32),
        ),
    )(x, w1d, w1s, b1_row)


_mesh = plsc.VectorSubcoreMesh(core_axis_name="c", subcore_axis_name="s")


@functools.partial(
    pl.kernel,
    out_type=jax.ShapeDtypeStruct((_N_EDGES, _HID), jnp.float32),
    mesh=_mesh,
    scratch_types=[
        pltpu.VMEM((_DEPTH * _B,), jnp.int32),
        pltpu.VMEM((_DEPTH * _B,), jnp.int32),
        pltpu.VMEM((_DEPTH * _B,), jnp.float32),
        pltpu.VMEM((_HID,), jnp.float32),
        pltpu.VMEM((_DEPTH, _B, _HID), jnp.float32),
        pltpu.VMEM((_DEPTH, _B, _HID), jnp.float32),
        [pltpu.SemaphoreType.DMA] * _DEPTH,
        [pltpu.SemaphoreType.DMA] * _DEPTH,
        pltpu.SemaphoreType.DMA,
    ],
)
def _sc_edge(pd_hbm, ps_hbm, dst_hbm, src_hbm, dist_hbm, wd_hbm, h_hbm,
             dsti, srci, distv, wdv, gd, gs, semd, sems, semw):
    wid = lax.axis_index("s") * _NC + lax.axis_index("c")
    base0 = wid * _EPT

    pltpu.sync_copy(wd_hbm, wdv)
    wdc = [wdv[pl.ds(c * 16, 16)] for c in range(_HID // 16)]

    def stage_group(c0, n):
        pltpu.sync_copy(dst_hbm.at[pl.ds(base0 + c0 * _B, n * _B)],
                        dsti.at[pl.ds(0, n * _B)])
        pltpu.sync_copy(src_hbm.at[pl.ds(base0 + c0 * _B, n * _B)],
                        srci.at[pl.ds(0, n * _B)])
        pltpu.sync_copy(dist_hbm.at[pl.ds(base0 + c0 * _B, n * _B)],
                        distv.at[pl.ds(0, n * _B)])

    def issue_gathers(j):
        idx_d = dsti.at[pl.ds(j * _B, _B)]
        idx_s = srci.at[pl.ds(j * _B, _B)]
        cpd = pltpu.async_copy(pd_hbm.at[idx_d], gd.at[j], semd[j])
        cps = pltpu.async_copy(ps_hbm.at[idx_s], gs.at[j], sems[j])
        return cpd, cps

    def compute_chunk(j):
        def grp(g, carry):
            d16 = distv[pl.ds((j * (_B // 16) + g) * 16, 16)]
            for l in range(16):
                i = g * 16 + l
                dv = d16[l]
                for c in range(_HID // 16):
                    sl = pl.ds(c * 16, 16)
                    gd[j, i, sl] = jnp.maximum(
                        gd[j, i, sl] + gs[j, i, sl] + dv * wdc[c], 0.0)
            return carry

        lax.fori_loop(0, _B // 16, grp, 0)

    def write_chunk(ci, j):
        return pltpu.async_copy(gd.at[j], h_hbm.at[pl.ds(base0 + ci * _B, _B)], semw)

    def group(g, carry):
        c0 = g * _DEPTH
        stage_group(c0, _DEPTH)
        pend = [issue_gathers(j) for j in range(_DEPTH)]
        writes = []
        for j in range(_DEPTH):
            cpd, cps = pend[j]
            cpd.wait()
            cps.wait()
            compute_chunk(j)
            writes.append(write_chunk(c0 + j, j))
        for w in writes:
            w.wait()
        return carry

    lax.fori_loop(0, _NGRP, group, 0)

    ci = _NGRP * _DEPTH
    stage_group(ci, 1)
    cpd, cps = issue_gathers(0)
    cpd.wait()
    cps.wait()
    compute_chunk(0)
    write_chunk(ci, 0).wait()


def _post_body(h_ref, w2_ref, b2_ref, out_ref):
    out = jnp.dot(h_ref[...], w2_ref[...], preferred_element_type=jnp.float32) + b2_ref[...]
    out_ref[...] = jnp.maximum(out, 0.0)


def _post(h, w2, b2_row):
    be = 4000
    grid = _N_EDGES // be
    return pl.pallas_call(
        _post_body,
        grid=(grid,),
        in_specs=[
            pl.BlockSpec((be, _HID), lambda i: (i, 0)),
            pl.BlockSpec((_HID, _EMB), lambda i: (0, 0)),
            pl.BlockSpec((1, _EMB), lambda i: (0, 0)),
        ],
        out_specs=pl.BlockSpec((be, _EMB), lambda i: (i, 0)),
        out_shape=jax.ShapeDtypeStruct((_N_EDGES, _EMB), jnp.float32),
    )(h, w2, b2_row)


def kernel(node_features, node_hidden_state, edge_index, distance, W1, b1, W2, b2):
    x = jnp.concatenate([node_features, node_hidden_state], axis=1)
    pd, ps = _proj(x, W1[0:130], W1[130:260], b1.reshape(1, _HID))
    h = _sc_edge(pd, ps, edge_index[1], edge_index[0], distance, W1[260])
    return _post(h, W2, b2.reshape(1, _EMB))


# speedup vs baseline: 1.4271x; 1.4271x over previous
import functools

import jax
import jax.numpy as jnp
from jax import lax
from jax.experimental import pallas as pl
from jax.experimental.pallas import tpu as pltpu
from jax.experimental.pallas import tpu_sc as plsc

_N_NODES = 10000
_N_EDGES = 320000
_HID = 128
_EMB = 16

_NC = 2
_NS = 16
_NW = _NC * _NS
_EPT = _N_EDGES // _NW
_B = 80
_NCH = _EPT // _B
_DEPTH = 4
_NGRP = _NCH // _DEPTH


def _proj_body(x_ref, w1d_ref, w1s_ref, b1_ref, pd_ref, ps_ref):
    x = x_ref[...]
    pd_ref[...] = jnp.dot(x, w1d_ref[...], preferred_element_type=jnp.float32) + b1_ref[...]
    ps_ref[...] = jnp.dot(x, w1s_ref[...], preferred_element_type=jnp.float32)


def _proj(x, w1d, w1s, b1_row):
    return pl.pallas_call(
        _proj_body,
        out_shape=(
            jax.ShapeDtypeStruct((_N_NODES, _HID), jnp.float32),
            jax.ShapeDtypeStruct((_N_NODES, _HID), jnp.float32),
        ),
    )(x, w1d, w1s, b1_row)


_mesh = plsc.VectorSubcoreMesh(core_axis_name="c", subcore_axis_name="s")


@functools.partial(
    pl.kernel,
    out_type=jax.ShapeDtypeStruct((_N_EDGES, _HID), jnp.float32),
    mesh=_mesh,
    scratch_types=[
        pltpu.VMEM((_DEPTH * _B,), jnp.int32),
        pltpu.VMEM((_DEPTH * _B,), jnp.int32),
        pltpu.VMEM((_DEPTH, _B, _HID), jnp.float32),
        pltpu.VMEM((_DEPTH, _B, _HID), jnp.float32),
        [pltpu.SemaphoreType.DMA] * _DEPTH,
        [pltpu.SemaphoreType.DMA] * _DEPTH,
        pltpu.SemaphoreType.DMA,
    ],
)
def _sc_edge(pd_hbm, ps_hbm, dst_hbm, src_hbm, h_hbm,
             dsti, srci, gd, gs, semd, sems, semw):
    wid = lax.axis_index("s") * _NC + lax.axis_index("c")
    base0 = wid * _EPT

    def stage_group(c0, n):
        pltpu.sync_copy(dst_hbm.at[pl.ds(base0 + c0 * _B, n * _B)],
                        dsti.at[pl.ds(0, n * _B)])
        pltpu.sync_copy(src_hbm.at[pl.ds(base0 + c0 * _B, n * _B)],
                        srci.at[pl.ds(0, n * _B)])

    def issue_gathers(j):
        idx_d = dsti.at[pl.ds(j * _B, _B)]
        idx_s = srci.at[pl.ds(j * _B, _B)]
        cpd = pltpu.async_copy(pd_hbm.at[idx_d], gd.at[j], semd[j])
        cps = pltpu.async_copy(ps_hbm.at[idx_s], gs.at[j], sems[j])
        return cpd, cps

    def compute_chunk(j):
        def row(i, carry):
            for c in range(_HID // 16):
                sl = pl.ds(c * 16, 16)
                plsc.addupdate(gd.at[j, i, sl], gs[j, i, sl])
            return carry

        lax.fori_loop(0, _B, row, 0)

    def write_chunk(ci, j):
        return pltpu.async_copy(gd.at[j], h_hbm.at[pl.ds(base0 + ci * _B, _B)], semw)

    def group(g, carry):
        c0 = g * _DEPTH
        stage_group(c0, _DEPTH)
        pend = [issue_gathers(j) for j in range(_DEPTH)]
        writes = []
        for j in range(_DEPTH):
            cpd, cps = pend[j]
            cpd.wait()
            cps.wait()
            compute_chunk(j)
            writes.append(write_chunk(c0 + j, j))
        for w in writes:
            w.wait()
        return carry

    lax.fori_loop(0, _NGRP, group, 0)

    ci = _NGRP * _DEPTH
    stage_group(ci, 1)
    cpd, cps = issue_gathers(0)
    cpd.wait()
    cps.wait()
    compute_chunk(0)
    write_chunk(ci, 0).wait()


def _post_body(h_ref, dist_ref, wd_ref, w2_ref, b2_ref, out_ref):
    h = jnp.maximum(h_ref[...] + dist_ref[...] * wd_ref[...], 0.0)
    out = jnp.dot(h, w2_ref[...], preferred_element_type=jnp.float32) + b2_ref[...]
    out_ref[...] = jnp.maximum(out, 0.0)


def _post(h, dist_col, wd_row, w2, b2_row):
    be = 4000
    grid = _N_EDGES // be
    return pl.pallas_call(
        _post_body,
        grid=(grid,),
        in_specs=[
            pl.BlockSpec((be, _HID), lambda i: (i, 0)),
            pl.BlockSpec((be, 1), lambda i: (i, 0)),
            pl.BlockSpec((1, _HID), lambda i: (0, 0)),
            pl.BlockSpec((_HID, _EMB), lambda i: (0, 0)),
            pl.BlockSpec((1, _EMB), lambda i: (0, 0)),
        ],
        out_specs=pl.BlockSpec((be, _EMB), lambda i: (i, 0)),
        out_shape=jax.ShapeDtypeStruct((_N_EDGES, _EMB), jnp.float32),
    )(h, dist_col, wd_row, w2, b2_row)


def kernel(node_features, node_hidden_state, edge_index, distance, W1, b1, W2, b2):
    x = jnp.concatenate([node_features, node_hidden_state], axis=1)
    pd, ps = _proj(x, W1[0:130], W1[130:260], b1.reshape(1, _HID))
    h = _sc_edge(pd, ps, edge_index[1], edge_index[0])
    return _post(
        h,
        distance.reshape(_N_EDGES, 1),
        W1[260].reshape(1, _HID),
        W2,
        b2.reshape(1, _EMB),
    )


# speedup vs baseline: 1.5498x; 1.0860x over previous
import functools

import jax
import jax.numpy as jnp
from jax import lax
from jax.experimental import pallas as pl
from jax.experimental.pallas import tpu as pltpu
from jax.experimental.pallas import tpu_sc as plsc

_N_NODES = 10000
_N_EDGES = 320000
_HID = 128
_EMB = 16

_NC = 2
_NS = 16
_NW = _NC * _NS
_EPT = _N_EDGES // _NW
_B = 80
_NCH = _EPT // _B
_DEPTH = 4
_NGRP = _NCH // _DEPTH


def _proj_body(x_ref, w1d_ref, w1s_ref, b1_ref, pd_ref, ps_ref):
    x = x_ref[...]
    pd_ref[...] = jnp.dot(x, w1d_ref[...], preferred_element_type=jnp.float32) + b1_ref[...]
    ps_ref[...] = jnp.dot(x, w1s_ref[...], preferred_element_type=jnp.float32)


def _proj(x, w1d, w1s, b1_row):
    return pl.pallas_call(
        _proj_body,
        out_shape=(
            jax.ShapeDtypeStruct((_N_NODES, _HID), jnp.float32),
            jax.ShapeDtypeStruct((_N_NODES, _HID), jnp.float32),
        ),
    )(x, w1d, w1s, b1_row)


_mesh = plsc.VectorSubcoreMesh(core_axis_name="c", subcore_axis_name="s")


@functools.partial(
    pl.kernel,
    out_type=jax.ShapeDtypeStruct((_N_EDGES, _HID), jnp.float32),
    mesh=_mesh,
    scratch_types=[
        pltpu.VMEM((_DEPTH * _B,), jnp.int32),
        pltpu.VMEM((_DEPTH * _B,), jnp.int32),
        pltpu.VMEM((_DEPTH, _B, _HID), jnp.float32),
        pltpu.VMEM((_DEPTH, _B, _HID), jnp.float32),
        [pltpu.SemaphoreType.DMA] * _DEPTH,
        [pltpu.SemaphoreType.DMA] * _DEPTH,
        pltpu.SemaphoreType.DMA,
    ],
)
def _sc_edge(pd_hbm, ps_hbm, dst_hbm, src_hbm, h_hbm,
             dsti, srci, gd, gs, semd, sems, semw):
    wid = lax.axis_index("s") * _NC + lax.axis_index("c")
    base0 = wid * _EPT

    def stage_group(c0, n):
        pltpu.sync_copy(dst_hbm.at[pl.ds(base0 + c0 * _B, n * _B)],
                        dsti.at[pl.ds(0, n * _B)])
        pltpu.sync_copy(src_hbm.at[pl.ds(base0 + c0 * _B, n * _B)],
                        srci.at[pl.ds(0, n * _B)])

    def issue_gathers(j):
        idx_d = dsti.at[pl.ds(j * _B, _B)]
        idx_s = srci.at[pl.ds(j * _B, _B)]
        cpd = pltpu.async_copy(pd_hbm.at[idx_d], gd.at[j], semd[j])
        cps = pltpu.async_copy(ps_hbm.at[idx_s], gs.at[j], sems[j])
        return cpd, cps

    def compute_chunk(j):
        def row(i, carry):
            for c in range(_HID // 16):
                sl = pl.ds(c * 16, 16)
                plsc.addupdate(gd.at[j, i, sl], gs[j, i, sl])
            return carry

        lax.fori_loop(0, _B, row, 0)

    def write_chunk(ci, j):
        return pltpu.async_copy(gd.at[j], h_hbm.at[pl.ds(base0 + ci * _B, _B)], semw)

    def group(g, carry):
        c0 = g * _DEPTH
        stage_group(c0, _DEPTH)
        pend = [issue_gathers(j) for j in range(_DEPTH)]
        writes = []
        for j in range(_DEPTH):
            cpd, cps = pend[j]
            cpd.wait()
            cps.wait()
            compute_chunk(j)
            writes.append(write_chunk(c0 + j, j))
        for w in writes:
            w.wait()
        return carry

    lax.fori_loop(0, _NGRP, group, 0)

    ci = _NGRP * _DEPTH
    stage_group(ci, 1)
    cpd, cps = issue_gathers(0)
    cpd.wait()
    cps.wait()
    compute_chunk(0)
    write_chunk(ci, 0).wait()


def _post_body(h_ref, dist_ref, wd_ref, w2_ref, b2_ref, out_ref):
    be = h_ref.shape[0]
    dr = dist_ref.shape[0]
    d = dist_ref[...]
    p1 = jax.lax.broadcast_in_dim(d, (dr, 128, 128), (0, 2)).reshape(be, 128)
    lane = jax.lax.broadcasted_iota(jnp.int32, (be, 128), 1)
    row = jax.lax.broadcasted_iota(jnp.int32, (be, 128), 0)
    dist_col = jnp.sum(jnp.where(lane == row % 128, p1, 0.0), axis=1,
                       keepdims=True)
    h = jnp.maximum(h_ref[...] + dist_col * wd_ref[...], 0.0)
    out = jnp.dot(h, w2_ref[...], preferred_element_type=jnp.float32) + b2_ref[...]
    out_ref[...] = jnp.maximum(out, 0.0)


def _post(h, dist2d, wd_row, w2, b2_row):
    be = 4096
    grid = pl.cdiv(_N_EDGES, be)
    dr = be // 128
    return pl.pallas_call(
        _post_body,
        grid=(grid,),
        in_specs=[
            pl.BlockSpec((be, _HID), lambda i: (i, 0)),
            pl.BlockSpec((dr, 128), lambda i: (i, 0)),
            pl.BlockSpec((1, _HID), lambda i: (0, 0)),
            pl.BlockSpec((_HID, _EMB), lambda i: (0, 0)),
            pl.BlockSpec((1, _EMB), lambda i: (0, 0)),
        ],
        out_specs=pl.BlockSpec((be, _EMB), lambda i: (i, 0)),
        out_shape=jax.ShapeDtypeStruct((_N_EDGES, _EMB), jnp.float32),
    )(h, dist2d, wd_row, w2, b2_row)


def kernel(node_features, node_hidden_state, edge_index, distance, W1, b1, W2, b2):
    x = jnp.concatenate([node_features, node_hidden_state], axis=1)
    pd, ps = _proj(x, W1[0:130], W1[130:260], b1.reshape(1, _HID))
    h = _sc_edge(pd, ps, edge_index[1], edge_index[0])
    return _post(
        h,
        distance.reshape(_N_EDGES // 128, 128),
        W1[260].reshape(1, _HID),
        W2,
        b2.reshape(1, _EMB),
    )


# speedup vs baseline: 1.9924x; 1.2856x over previous
import functools

import jax
import jax.numpy as jnp
from jax import lax
from jax.experimental import pallas as pl
from jax.experimental.pallas import tpu as pltpu
from jax.experimental.pallas import tpu_sc as plsc

_N_NODES = 10000
_N_EDGES = 320000
_HID = 128
_EMB = 16

_NC = 2
_NS = 16
_NW = _NC * _NS
_EPT = _N_EDGES // _NW
_B = 80
_NCH = _EPT // _B
_DEPTH = 4
_NGRP = _NCH // _DEPTH


def _proj_body(x_ref, w1d_ref, w1s_ref, b1_ref, pd_ref, ps_ref):
    x = x_ref[...]
    pd_ref[...] = jnp.dot(x, w1d_ref[...], preferred_element_type=jnp.float32) + b1_ref[...]
    ps_ref[...] = jnp.dot(x, w1s_ref[...], preferred_element_type=jnp.float32)


def _proj(x, w1d, w1s, b1_row):
    return pl.pallas_call(
        _proj_body,
        out_shape=(
            jax.ShapeDtypeStruct((_N_NODES, _HID), jnp.float32),
            jax.ShapeDtypeStruct((_N_NODES, _HID), jnp.float32),
        ),
    )(x, w1d, w1s, b1_row)


_mesh = plsc.VectorSubcoreMesh(core_axis_name="c", subcore_axis_name="s")


@functools.partial(
    pl.kernel,
    out_type=jax.ShapeDtypeStruct((_N_EDGES, _HID), jnp.float32),
    mesh=_mesh,
    scratch_types=[
        pltpu.VMEM((_DEPTH * _B,), jnp.int32),
        pltpu.VMEM((_DEPTH * _B,), jnp.int32),
        pltpu.VMEM((_DEPTH, _B, _HID), jnp.float32),
        pltpu.VMEM((_DEPTH, _B, _HID), jnp.float32),
        [pltpu.SemaphoreType.DMA] * _DEPTH,
        [pltpu.SemaphoreType.DMA] * _DEPTH,
        pltpu.SemaphoreType.DMA,
    ],
)
def _sc_edge(pd_hbm, ps_hbm, dst_hbm, src_hbm, h_hbm,
             dsti, srci, gd, gs, semd, sems, semw):
    wid = lax.axis_index("s") * _NC + lax.axis_index("c")
    base0 = wid * _EPT

    def stage_group(c0, n):
        pltpu.sync_copy(dst_hbm.at[pl.ds(base0 + c0 * _B, n * _B)],
                        dsti.at[pl.ds(0, n * _B)])
        pltpu.sync_copy(src_hbm.at[pl.ds(base0 + c0 * _B, n * _B)],
                        srci.at[pl.ds(0, n * _B)])

    def issue_gathers(j):
        idx_d = dsti.at[pl.ds(j * _B, _B)]
        idx_s = srci.at[pl.ds(j * _B, _B)]
        cpd = pltpu.async_copy(pd_hbm.at[idx_d], gd.at[j], semd[j])
        cps = pltpu.async_copy(ps_hbm.at[idx_s], gs.at[j], sems[j])
        return cpd, cps

    def compute_chunk(j):
        def row(i, carry):
            for c in range(_HID // 16):
                sl = pl.ds(c * 16, 16)
                plsc.addupdate(gd.at[j, i, sl], gs[j, i, sl])
            return carry

        lax.fori_loop(0, _B, row, 0)

    def write_chunk(ci, j):
        return pltpu.async_copy(gd.at[j], h_hbm.at[pl.ds(base0 + ci * _B, _B)], semw)

    def group(g, carry):
        c0 = g * _DEPTH
        stage_group(c0, _DEPTH)
        pend = [issue_gathers(j) for j in range(_DEPTH)]
        writes = []
        for j in range(_DEPTH):
            cpd, cps = pend[j]
            cpd.wait()
            cps.wait()
            compute_chunk(j)
            writes.append(write_chunk(c0 + j, j))
        for w in writes:
            w.wait()
        return carry

    lax.fori_loop(0, _NGRP, group, 0)

    ci = _NGRP * _DEPTH
    stage_group(ci, 1)
    cpd, cps = issue_gathers(0)
    cpd.wait()
    cps.wait()
    compute_chunk(0)
    write_chunk(ci, 0).wait()


def _post_body(h_ref, dist_ref, wd_ref, w2_ref, b2_ref, out_ref):
    be = h_ref.shape[0]
    dr = dist_ref.shape[0]
    d = dist_ref[...]
    p1 = jax.lax.broadcast_in_dim(d, (dr, 128, 128), (0, 2)).reshape(be, 128)
    lane = jax.lax.broadcasted_iota(jnp.int32, (be, 128), 1)
    row = jax.lax.broadcasted_iota(jnp.int32, (be, 128), 0)
    dist_col = jnp.sum(jnp.where(lane == row % 128, p1, 0.0), axis=1,
                       keepdims=True)
    h = jnp.maximum(h_ref[...] + dist_col * wd_ref[...], 0.0)
    out = jnp.dot(h, w2_ref[...], preferred_element_type=jnp.float32) + b2_ref[...]
    out_ref[...] = jnp.maximum(out, 0.0).T


def _post(h, dist2d, wd_row, w2, b2_row):
    be = 8192
    grid = pl.cdiv(_N_EDGES, be)
    dr = be // 128
    return pl.pallas_call(
        _post_body,
        grid=(grid,),
        in_specs=[
            pl.BlockSpec((be, _HID), lambda i: (i, 0)),
            pl.BlockSpec((dr, 128), lambda i: (i, 0)),
            pl.BlockSpec((1, _HID), lambda i: (0, 0)),
            pl.BlockSpec((_HID, _EMB), lambda i: (0, 0)),
            pl.BlockSpec((1, _EMB), lambda i: (0, 0)),
        ],
        out_specs=pl.BlockSpec((_EMB, be), lambda i: (0, i)),
        out_shape=jax.ShapeDtypeStruct((_EMB, _N_EDGES), jnp.float32),
    )(h, dist2d, wd_row, w2, b2_row)


def kernel(node_features, node_hidden_state, edge_index, distance, W1, b1, W2, b2):
    x = jnp.concatenate([node_features, node_hidden_state], axis=1)
    pd, ps = _proj(x, W1[0:130], W1[130:260], b1.reshape(1, _HID))
    h = _sc_edge(pd, ps, edge_index[1], edge_index[0])
    out_t = _post(
        h,
        distance.reshape(_N_EDGES // 128, 128),
        W1[260].reshape(1, _HID),
        W2,
        b2.reshape(1, _EMB),
    )
    return out_t.T


# speedup vs baseline: 2.0011x; 1.0044x over previous
import functools

import jax
import jax.numpy as jnp
from jax import lax
from jax.experimental import pallas as pl
from jax.experimental.pallas import tpu as pltpu
from jax.experimental.pallas import tpu_sc as plsc

_N_NODES = 10000
_N_EDGES = 320000
_HID = 128
_EMB = 16

_NC = 2
_NS = 16
_NW = _NC * _NS
_B = 80
_DEPTH = 4
_SEG0 = 192000
_SEG1 = _N_EDGES - _SEG0


def _proj_body(x_ref, w1d_ref, w1s_ref, b1_ref, pd_ref, ps_ref):
    x = x_ref[...]
    pd_ref[...] = jnp.dot(x, w1d_ref[...], preferred_element_type=jnp.float32) + b1_ref[...]
    ps_ref[...] = jnp.dot(x, w1s_ref[...], preferred_element_type=jnp.float32)


def _proj(x, w1d, w1s, b1_row):
    return pl.pallas_call(
        _proj_body,
        out_shape=(
            jax.ShapeDtypeStruct((_N_NODES, _HID), jnp.float32),
            jax.ShapeDtypeStruct((_N_NODES, _HID), jnp.float32),
        ),
    )(x, w1d, w1s, b1_row)


_mesh = plsc.VectorSubcoreMesh(core_axis_name="c", subcore_axis_name="s")


def _make_sc_edge(n_edges):
    ept = n_edges // _NW
    nch = ept // _B
    ngrp = nch // _DEPTH
    rem = nch % _DEPTH

    @functools.partial(
        pl.kernel,
        out_type=jax.ShapeDtypeStruct((n_edges, _HID), jnp.float32),
        mesh=_mesh,
        scratch_types=[
            pltpu.VMEM((_DEPTH * _B,), jnp.int32),
            pltpu.VMEM((_DEPTH * _B,), jnp.int32),
            pltpu.VMEM((_DEPTH, _B, _HID), jnp.float32),
            pltpu.VMEM((_DEPTH, _B, _HID), jnp.float32),
            [pltpu.SemaphoreType.DMA] * _DEPTH,
            [pltpu.SemaphoreType.DMA] * _DEPTH,
            pltpu.SemaphoreType.DMA,
        ],
    )
    def sc_edge(pd_hbm, ps_hbm, dst_hbm, src_hbm, h_hbm,
                dsti, srci, gd, gs, semd, sems, semw):
        wid = lax.axis_index("s") * _NC + lax.axis_index("c")
        base0 = wid * ept

        def stage_group(c0, n):
            pltpu.sync_copy(dst_hbm.at[pl.ds(base0 + c0 * _B, n * _B)],
                            dsti.at[pl.ds(0, n * _B)])
            pltpu.sync_copy(src_hbm.at[pl.ds(base0 + c0 * _B, n * _B)],
                            srci.at[pl.ds(0, n * _B)])

        def issue_gathers(j):
            cpd = pltpu.async_copy(pd_hbm.at[dsti.at[pl.ds(j * _B, _B)]],
                                   gd.at[j], semd[j])
            cps = pltpu.async_copy(ps_hbm.at[srci.at[pl.ds(j * _B, _B)]],
                                   gs.at[j], sems[j])
            return cpd, cps

        def compute_chunk(j):
            def row(i, carry):
                for c in range(_HID // 16):
                    sl = pl.ds(c * 16, 16)
                    plsc.addupdate(gd.at[j, i, sl], gs[j, i, sl])
                return carry

            lax.fori_loop(0, _B, row, 0)

        def write_chunk(ci, j):
            return pltpu.async_copy(gd.at[j],
                                    h_hbm.at[pl.ds(base0 + ci * _B, _B)], semw)

        def run_group(c0, n):
            stage_group(c0, n)
            pend = [issue_gathers(j) for j in range(n)]
            writes = []
            for j in range(n):
                cpd, cps = pend[j]
                cpd.wait()
                cps.wait()
                compute_chunk(j)
                writes.append(write_chunk(c0 + j, j))
            for w in writes:
                w.wait()

        def group(g, carry):
            run_group(g * _DEPTH, _DEPTH)
            return carry

        lax.fori_loop(0, ngrp, group, 0)
        if rem:
            run_group(ngrp * _DEPTH, rem)

    return sc_edge


_sc_edge0 = _make_sc_edge(_SEG0)
_sc_edge1 = _make_sc_edge(_SEG1)


def _post_body(h_ref, dist_ref, wd_ref, w2_ref, b2_ref, out_ref):
    be = h_ref.shape[0]
    dr = dist_ref.shape[0]
    d = dist_ref[...]
    p1 = jax.lax.broadcast_in_dim(d, (dr, 128, 128), (0, 2)).reshape(be, 128)
    lane = jax.lax.broadcasted_iota(jnp.int32, (be, 128), 1)
    row = jax.lax.broadcasted_iota(jnp.int32, (be, 128), 0)
    dist_col = jnp.sum(jnp.where(lane == row % 128, p1, 0.0), axis=1,
                       keepdims=True)
    h = jnp.maximum(h_ref[...] + dist_col * wd_ref[...], 0.0)
    out = jnp.dot(h, w2_ref[...], preferred_element_type=jnp.float32) + b2_ref[...]
    out_ref[...] = jnp.maximum(out, 0.0).T


def _post(h, dist2d, wd_row, w2, b2_row, n_edges):
    be = 8192
    grid = pl.cdiv(n_edges, be)
    dr = be // 128
    return pl.pallas_call(
        _post_body,
        grid=(grid,),
        in_specs=[
            pl.BlockSpec((be, _HID), lambda i: (i, 0)),
            pl.BlockSpec((dr, 128), lambda i: (i, 0)),
            pl.BlockSpec((1, _HID), lambda i: (0, 0)),
            pl.BlockSpec((_HID, _EMB), lambda i: (0, 0)),
            pl.BlockSpec((1, _EMB), lambda i: (0, 0)),
        ],
        out_specs=pl.BlockSpec((_EMB, be), lambda i: (0, i)),
        out_shape=jax.ShapeDtypeStruct((_EMB, n_edges), jnp.float32),
    )(h, dist2d, wd_row, w2, b2_row)


def kernel(node_features, node_hidden_state, edge_index, distance, W1, b1, W2, b2):
    x = jnp.concatenate([node_features, node_hidden_state], axis=1)
    pd, ps = _proj(x, W1[0:130], W1[130:260], b1.reshape(1, _HID))
    dst = edge_index[1]
    src = edge_index[0]
    wd_row = W1[260].reshape(1, _HID)
    b2_row = b2.reshape(1, _EMB)
    h0 = _sc_edge0(pd, ps, dst[:_SEG0], src[:_SEG0])
    h1 = _sc_edge1(pd, ps, dst[_SEG0:], src[_SEG0:])
    o0 = _post(h0, distance[:_SEG0].reshape(_SEG0 // 128, 128), wd_row, W2,
               b2_row, _SEG0)
    o1 = _post(h1, distance[_SEG0:].reshape(_SEG1 // 128, 128), wd_row, W2,
               b2_row, _SEG1)
    return jnp.concatenate([o0, o1], axis=1).T


# speedup vs baseline: 2.0920x; 1.0454x over previous
import functools

import jax
import jax.numpy as jnp
from jax import lax
from jax.experimental import pallas as pl
from jax.experimental.pallas import tpu as pltpu
from jax.experimental.pallas import tpu_sc as plsc

_N_NODES = 10000
_N_EDGES = 320000
_HID = 128
_EMB = 16

_NC = 2
_NS = 16
_NW = _NC * _NS
_B = 80
_DEPTH = 4
_SEG0 = 128000
_SEG1 = _N_EDGES - _SEG0


def _proj_body(nf_ref, nh_ref, w1_ref, b1_ref, pd_ref, ps_ref):
    nf = nf_ref[...]
    nh = nh_ref[...]
    pd_ref[...] = (
        jnp.dot(nf, w1_ref[0:2], preferred_element_type=jnp.float32)
        + jnp.dot(nh, w1_ref[2:130], preferred_element_type=jnp.float32)
        + b1_ref[...]
    )
    ps_ref[...] = (
        jnp.dot(nf, w1_ref[130:132], preferred_element_type=jnp.float32)
        + jnp.dot(nh, w1_ref[132:260], preferred_element_type=jnp.float32)
    )


def _proj(nf, nh, w1, b1_row):
    return pl.pallas_call(
        _proj_body,
        out_shape=(
            jax.ShapeDtypeStruct((_N_NODES, _HID), jnp.float32),
            jax.ShapeDtypeStruct((_N_NODES, _HID), jnp.float32),
        ),
    )(nf, nh, w1, b1_row)


_mesh = plsc.VectorSubcoreMesh(core_axis_name="c", subcore_axis_name="s")


def _make_sc_edge(n_edges, seg_base):
    ept = n_edges // _NW
    nch = ept // _B
    ngrp = nch // _DEPTH
    rem = nch % _DEPTH

    @functools.partial(
        pl.kernel,
        out_type=jax.ShapeDtypeStruct((n_edges, _HID), jnp.float32),
        mesh=_mesh,
        scratch_types=[
            pltpu.VMEM((_DEPTH * _B,), jnp.int32),
            pltpu.VMEM((_DEPTH * _B,), jnp.int32),
            pltpu.VMEM((_DEPTH, _B, _HID), jnp.float32),
            pltpu.VMEM((_DEPTH, _B, _HID), jnp.float32),
            [pltpu.SemaphoreType.DMA] * _DEPTH,
            [pltpu.SemaphoreType.DMA] * _DEPTH,
            pltpu.SemaphoreType.DMA,
        ],
    )
    def sc_edge(pd_hbm, ps_hbm, ei_hbm, h_hbm,
                dsti, srci, gd, gs, semd, sems, semw):
        wid = lax.axis_index("s") * _NC + lax.axis_index("c")
        base0 = wid * ept
        doff = _N_EDGES + seg_base
        soff = seg_base

        def stage_group(c0, n):
            pltpu.sync_copy(ei_hbm.at[pl.ds(doff + base0 + c0 * _B, n * _B)],
                            dsti.at[pl.ds(0, n * _B)])
            pltpu.sync_copy(ei_hbm.at[pl.ds(soff + base0 + c0 * _B, n * _B)],
                            srci.at[pl.ds(0, n * _B)])

        def issue_gathers(j):
            cpd = pltpu.async_copy(pd_hbm.at[dsti.at[pl.ds(j * _B, _B)]],
                                   gd.at[j], semd[j])
            cps = pltpu.async_copy(ps_hbm.at[srci.at[pl.ds(j * _B, _B)]],
                                   gs.at[j], sems[j])
            return cpd, cps

        def compute_chunk(j):
            def row(i, carry):
                for c in range(_HID // 16):
                    sl = pl.ds(c * 16, 16)
                    plsc.addupdate(gd.at[j, i, sl], gs[j, i, sl])
                return carry

            lax.fori_loop(0, _B, row, 0)

        def write_chunk(ci, j):
            return pltpu.async_copy(gd.at[j],
                                    h_hbm.at[pl.ds(base0 + ci * _B, _B)], semw)

        def run_group(c0, n):
            stage_group(c0, n)
            pend = [issue_gathers(j) for j in range(n)]
            writes = []
            for j in range(n):
                cpd, cps = pend[j]
                cpd.wait()
                cps.wait()
                compute_chunk(j)
                writes.append(write_chunk(c0 + j, j))
            for w in writes:
                w.wait()

        def group(g, carry):
            run_group(g * _DEPTH, _DEPTH)
            return carry

        lax.fori_loop(0, ngrp, group, 0)
        if rem:
            run_group(ngrp * _DEPTH, rem)

    return sc_edge


_sc_edge0 = _make_sc_edge(_SEG0, 0)
_sc_edge1 = _make_sc_edge(_SEG1, _SEG0)


def _post_body(h_ref, dist_ref, wd_ref, w2_ref, b2_ref, out_ref):
    be = h_ref.shape[0]
    dr = dist_ref.shape[0]
    d = dist_ref[...]
    p1 = jax.lax.broadcast_in_dim(d, (dr, 128, 128), (0, 2)).reshape(be, 128)
    lane = jax.lax.broadcasted_iota(jnp.int32, (be, 128), 1)
    row = jax.lax.broadcasted_iota(jnp.int32, (be, 128), 0)
    dist_col = jnp.sum(jnp.where(lane == row % 128, p1, 0.0), axis=1,
                       keepdims=True)
    h = jnp.maximum(h_ref[...] + dist_col * wd_ref[...], 0.0)
    out = jnp.dot(h, w2_ref[...], preferred_element_type=jnp.float32) + b2_ref[...]
    out_ref[...] = jnp.maximum(out, 0.0).T


def _post(h, dist2d, wd_row, w2, b2_row, n_edges):
    be = 8192
    grid = pl.cdiv(n_edges, be)
    dr = be // 128
    return pl.pallas_call(
        _post_body,
        grid=(grid,),
        in_specs=[
            pl.BlockSpec((be, _HID), lambda i: (i, 0)),
            pl.BlockSpec((dr, 128), lambda i: (i, 0)),
            pl.BlockSpec((1, _HID), lambda i: (0, 0)),
            pl.BlockSpec((_HID, _EMB), lambda i: (0, 0)),
            pl.BlockSpec((1, _EMB), lambda i: (0, 0)),
        ],
        out_specs=pl.BlockSpec((_EMB, be), lambda i: (0, i)),
        out_shape=jax.ShapeDtypeStruct((_EMB, n_edges), jnp.float32),
    )(h, dist2d, wd_row, w2, b2_row)


def kernel(node_features, node_hidden_state, edge_index, distance, W1, b1, W2, b2):
    pd, ps = _proj(node_features, node_hidden_state, W1, b1.reshape(1, _HID))
    ei1d = edge_index.reshape(2 * _N_EDGES)
    wd_row = W1[260].reshape(1, _HID)
    b2_row = b2.reshape(1, _EMB)
    h0 = _sc_edge0(pd, ps, ei1d)
    h1 = _sc_edge1(pd, ps, ei1d)
    o0 = _post(h0, distance[:_SEG0].reshape(_SEG0 // 128, 128), wd_row, W2,
               b2_row, _SEG0)
    o1 = _post(h1, distance[_SEG0:].reshape(_SEG1 // 128, 128), wd_row, W2,
               b2_row, _SEG1)
    return jnp.concatenate([o0, o1], axis=1).T


# speedup vs baseline: 2.1054x; 1.0064x over previous
import functools

import jax
import jax.numpy as jnp
from jax import lax
from jax.experimental import pallas as pl
from jax.experimental.pallas import tpu as pltpu
from jax.experimental.pallas import tpu_sc as plsc

_N_NODES = 10000
_N_EDGES = 320000
_HID = 128
_EMB = 16

_NC = 2
_NS = 16
_NW = _NC * _NS
_B = 80
_DEPTH = 4
_SEGS = (128000, 128000, 64000)


def _proj_body(nf_ref, nh_ref, w1_ref, b1_ref, pd_ref, ps_ref):
    nf = nf_ref[...]
    nh = nh_ref[...]
    pd_ref[...] = (
        jnp.dot(nf, w1_ref[0:2], preferred_element_type=jnp.float32)
        + jnp.dot(nh, w1_ref[2:130], preferred_element_type=jnp.float32)
        + b1_ref[...]
    )
    ps_ref[...] = (
        jnp.dot(nf, w1_ref[130:132], preferred_element_type=jnp.float32)
        + jnp.dot(nh, w1_ref[132:260], preferred_element_type=jnp.float32)
    )


def _proj(nf, nh, w1, b1_row):
    return pl.pallas_call(
        _proj_body,
        out_shape=(
            jax.ShapeDtypeStruct((_N_NODES, _HID), jnp.float32),
            jax.ShapeDtypeStruct((_N_NODES, _HID), jnp.float32),
        ),
    )(nf, nh, w1, b1_row)


_mesh = plsc.VectorSubcoreMesh(core_axis_name="c", subcore_axis_name="s")


def _make_sc_edge(n_edges, seg_base):
    ept = n_edges // _NW
    nch = ept // _B
    ngrp = nch // _DEPTH
    rem = nch % _DEPTH

    @functools.partial(
        pl.kernel,
        out_type=jax.ShapeDtypeStruct((n_edges, _HID), jnp.float32),
        mesh=_mesh,
        scratch_types=[
            pltpu.VMEM((_DEPTH * _B,), jnp.int32),
            pltpu.VMEM((_DEPTH * _B,), jnp.int32),
            pltpu.VMEM((_DEPTH, _B, _HID), jnp.float32),
            pltpu.VMEM((_DEPTH, _B, _HID), jnp.float32),
            [pltpu.SemaphoreType.DMA] * _DEPTH,
            [pltpu.SemaphoreType.DMA] * _DEPTH,
            pltpu.SemaphoreType.DMA,
        ],
    )
    def sc_edge(pd_hbm, ps_hbm, ei_hbm, h_hbm,
                dsti, srci, gd, gs, semd, sems, semw):
        wid = lax.axis_index("s") * _NC + lax.axis_index("c")
        base0 = wid * ept
        doff = _N_EDGES + seg_base
        soff = seg_base

        def stage_group(c0, n):
            pltpu.sync_copy(ei_hbm.at[pl.ds(doff + base0 + c0 * _B, n * _B)],
                            dsti.at[pl.ds(0, n * _B)])
            pltpu.sync_copy(ei_hbm.at[pl.ds(soff + base0 + c0 * _B, n * _B)],
                            srci.at[pl.ds(0, n * _B)])

        def issue_gathers(j):
            cpd = pltpu.async_copy(pd_hbm.at[dsti.at[pl.ds(j * _B, _B)]],
                                   gd.at[j], semd[j])
            cps = pltpu.async_copy(ps_hbm.at[srci.at[pl.ds(j * _B, _B)]],
                                   gs.at[j], sems[j])
            return cpd, cps

        def compute_chunk(j):
            def row(i, carry):
                for c in range(_HID // 16):
                    sl = pl.ds(c * 16, 16)
                    plsc.addupdate(gd.at[j, i, sl], gs[j, i, sl])
                return carry

            lax.fori_loop(0, _B, row, 0)

        def write_chunk(ci, j):
            return pltpu.async_copy(gd.at[j],
                                    h_hbm.at[pl.ds(base0 + ci * _B, _B)], semw)

        def run_group(c0, n):
            stage_group(c0, n)
            pend = [issue_gathers(j) for j in range(n)]
            writes = []
            for j in range(n):
                cpd, cps = pend[j]
                cpd.wait()
                cps.wait()
                compute_chunk(j)
                writes.append(write_chunk(c0 + j, j))
            for w in writes:
                w.wait()

        def group(g, carry):
            run_group(g * _DEPTH, _DEPTH)
            return carry

        lax.fori_loop(0, ngrp, group, 0)
        if rem:
            run_group(ngrp * _DEPTH, rem)

    return sc_edge


_sc_edges = []
_base = 0
for _n in _SEGS:
    _sc_edges.append(_make_sc_edge(_n, _base))
    _base += _n


def _post_body(h_ref, dist_ref, wd_ref, w2_ref, b2_ref, out_ref):
    be = h_ref.shape[0]
    dr = dist_ref.shape[0]
    d = dist_ref[...]
    p1 = jax.lax.broadcast_in_dim(d, (dr, 128, 128), (0, 2)).reshape(be, 128)
    lane = jax.lax.broadcasted_iota(jnp.int32, (be, 128), 1)
    row = jax.lax.broadcasted_iota(jnp.int32, (be, 128), 0)
    dist_col = jnp.sum(jnp.where(lane == row % 128, p1, 0.0), axis=1,
                       keepdims=True)
    h = jnp.maximum(h_ref[...] + dist_col * wd_ref[...], 0.0)
    out = jnp.dot(h, w2_ref[...], preferred_element_type=jnp.float32) + b2_ref[...]
    out_ref[...] = jnp.maximum(out, 0.0).T


def _post(h, dist2d, wd_row, w2, b2_row, n_edges):
    be = 8192
    grid = pl.cdiv(n_edges, be)
    dr = be // 128
    return pl.pallas_call(
        _post_body,
        grid=(grid,),
        in_specs=[
            pl.BlockSpec((be, _HID), lambda i: (i, 0)),
            pl.BlockSpec((dr, 128), lambda i: (i, 0)),
            pl.BlockSpec((1, _HID), lambda i: (0, 0)),
            pl.BlockSpec((_HID, _EMB), lambda i: (0, 0)),
            pl.BlockSpec((1, _EMB), lambda i: (0, 0)),
        ],
        out_specs=pl.BlockSpec((_EMB, be), lambda i: (0, i)),
        out_shape=jax.ShapeDtypeStruct((_EMB, n_edges), jnp.float32),
    )(h, dist2d, wd_row, w2, b2_row)


def kernel(node_features, node_hidden_state, edge_index, distance, W1, b1, W2, b2):
    pd, ps = _proj(node_features, node_hidden_state, W1, b1.reshape(1, _HID))
    ei1d = edge_index.reshape(2 * _N_EDGES)
    wd_row = W1[260].reshape(1, _HID)
    b2_row = b2.reshape(1, _EMB)
    hs = [sc(pd, ps, ei1d) for sc in _sc_edges]
    outs = []
    base = 0
    for h, n in zip(hs, _SEGS):
        outs.append(_post(h, distance[base:base + n].reshape(n // 128, 128),
                          wd_row, W2, b2_row, n))
        base += n
    return jnp.concatenate(outs, axis=1).T


# speedup vs baseline: 2.2884x; 1.0869x over previous
import functools

import jax
import jax.numpy as jnp
from jax import lax
from jax.experimental import pallas as pl
from jax.experimental.pallas import tpu as pltpu
from jax.experimental.pallas import tpu_sc as plsc

_N_NODES = 10000
_N_EDGES = 320000
_HID = 128
_EMB = 16

_NC = 2
_NS = 16
_NW = _NC * _NS
_B = 80
_DEPTH = 5
_SEGS = (128000, 128000, 64000)


def _proj_body(nf_ref, nh_ref, w1_ref, b1_ref, pd_ref, ps_ref):
    nf = nf_ref[...]
    nh = nh_ref[...]
    pd_ref[...] = (
        jnp.dot(nf, w1_ref[0:2], preferred_element_type=jnp.float32)
        + jnp.dot(nh, w1_ref[2:130], preferred_element_type=jnp.float32)
        + b1_ref[...]
    )
    ps_ref[...] = (
        jnp.dot(nf, w1_ref[130:132], preferred_element_type=jnp.float32)
        + jnp.dot(nh, w1_ref[132:260], preferred_element_type=jnp.float32)
    )


def _proj(nf, nh, w1, b1_row):
    return pl.pallas_call(
        _proj_body,
        out_shape=(
            jax.ShapeDtypeStruct((_N_NODES, _HID), jnp.float32),
            jax.ShapeDtypeStruct((_N_NODES, _HID), jnp.float32),
        ),
    )(nf, nh, w1, b1_row)


_mesh = plsc.VectorSubcoreMesh(core_axis_name="c", subcore_axis_name="s")


def _make_sc_edge(n_edges, seg_base):
    ept = n_edges // _NW
    nch = ept // _B
    ngrp = nch // _DEPTH
    rem = nch % _DEPTH

    @functools.partial(
        pl.kernel,
        out_type=jax.ShapeDtypeStruct((n_edges, _HID), jnp.float32),
        mesh=_mesh,
        scratch_types=[
            pltpu.VMEM((_DEPTH * _B,), jnp.int32),
            pltpu.VMEM((_DEPTH * _B,), jnp.int32),
            pltpu.VMEM((_DEPTH, _B, _HID), jnp.float32),
            pltpu.VMEM((_DEPTH, _B, _HID), jnp.float32),
            [pltpu.SemaphoreType.DMA] * _DEPTH,
            [pltpu.SemaphoreType.DMA] * _DEPTH,
            pltpu.SemaphoreType.DMA,
        ],
    )
    def sc_edge(pd_hbm, ps_hbm, ei_hbm, h_hbm,
                dsti, srci, gd, gs, semd, sems, semw):
        wid = lax.axis_index("s") * _NC + lax.axis_index("c")
        base0 = wid * ept
        doff = _N_EDGES + seg_base
        soff = seg_base

        def stage_group(c0, n):
            pltpu.sync_copy(ei_hbm.at[pl.ds(doff + base0 + c0 * _B, n * _B)],
                            dsti.at[pl.ds(0, n * _B)])
            pltpu.sync_copy(ei_hbm.at[pl.ds(soff + base0 + c0 * _B, n * _B)],
                            srci.at[pl.ds(0, n * _B)])

        def issue_gathers(j):
            cpd = pltpu.async_copy(pd_hbm.at[dsti.at[pl.ds(j * _B, _B)]],
                                   gd.at[j], semd[j])
            cps = pltpu.async_copy(ps_hbm.at[srci.at[pl.ds(j * _B, _B)]],
                                   gs.at[j], sems[j])
            return cpd, cps

        def compute_chunk(j):
            def row(i, carry):
                for c in range(_HID // 16):
                    sl = pl.ds(c * 16, 16)
                    plsc.addupdate(gd.at[j, i, sl], gs[j, i, sl])
                return carry

            lax.fori_loop(0, _B, row, 0)

        def write_chunk(ci, j):
            return pltpu.async_copy(gd.at[j],
                                    h_hbm.at[pl.ds(base0 + ci * _B, _B)], semw)

        def run_group(c0, n):
            stage_group(c0, n)
            pend = [issue_gathers(j) for j in range(n)]
            writes = []
            for j in range(n):
                cpd, cps = pend[j]
                cpd.wait()
                cps.wait()
                compute_chunk(j)
                writes.append(write_chunk(c0 + j, j))
            for w in writes:
                w.wait()

        def group(g, carry):
            run_group(g * _DEPTH, _DEPTH)
            return carry

        lax.fori_loop(0, ngrp, group, 0)
        if rem:
            run_group(ngrp * _DEPTH, rem)

    return sc_edge


_sc_edges = []
_base = 0
for _n in _SEGS:
    _sc_edges.append(_make_sc_edge(_n, _base))
    _base += _n


def _post_body(h_ref, dist_ref, wd_ref, w2_ref, b2_ref, out_ref):
    be = h_ref.shape[0]
    dr = dist_ref.shape[0]
    d = dist_ref[...]
    p1 = jax.lax.broadcast_in_dim(d, (dr, 128, 128), (0, 2)).reshape(be, 128)
    lane = jax.lax.broadcasted_iota(jnp.int32, (be, 128), 1)
    row = jax.lax.broadcasted_iota(jnp.int32, (be, 128), 0)
    dist_col = jnp.sum(jnp.where(lane == row % 128, p1, 0.0), axis=1,
                       keepdims=True)
    h = jnp.maximum(h_ref[...] + dist_col * wd_ref[...], 0.0)
    out = jnp.dot(h, w2_ref[...], preferred_element_type=jnp.float32) + b2_ref[...]
    out_ref[...] = jnp.maximum(out, 0.0).T


def _post(h, dist2d, wd_row, w2, b2_row, n_edges):
    be = 8192
    grid = pl.cdiv(n_edges, be)
    dr = be // 128
    return pl.pallas_call(
        _post_body,
        grid=(grid,),
        in_specs=[
            pl.BlockSpec((be, _HID), lambda i: (i, 0)),
            pl.BlockSpec((dr, 128), lambda i: (i, 0)),
            pl.BlockSpec((1, _HID), lambda i: (0, 0)),
            pl.BlockSpec((_HID, _EMB), lambda i: (0, 0)),
            pl.BlockSpec((1, _EMB), lambda i: (0, 0)),
        ],
        out_specs=pl.BlockSpec((_EMB, be), lambda i: (0, i)),
        out_shape=jax.ShapeDtypeStruct((_EMB, n_edges), jnp.float32),
    )(h, dist2d, wd_row, w2, b2_row)


def kernel(node_features, node_hidden_state, edge_index, distance, W1, b1, W2, b2):
    pd, ps = _proj(node_features, node_hidden_state, W1, b1.reshape(1, _HID))
    ei1d = edge_index.reshape(2 * _N_EDGES)
    wd_row = W1[260].reshape(1, _HID)
    b2_row = b2.reshape(1, _EMB)
    hs = [sc(pd, ps, ei1d) for sc in _sc_edges]
    outs = []
    base = 0
    for h, n in zip(hs, _SEGS):
        outs.append(_post(h, distance[base:base + n].reshape(n // 128, 128),
                          wd_row, W2, b2_row, n))
        base += n
    return jnp.concatenate(outs, axis=1).T


# speedup vs baseline: 2.3150x; 1.0117x over previous
import functools

import jax
import jax.numpy as jnp
from jax import lax
from jax.experimental import pallas as pl
from jax.experimental.pallas import tpu as pltpu
from jax.experimental.pallas import tpu_sc as plsc

_N_NODES = 10000
_N_EDGES = 320000
_HID = 128
_EMB = 16

_NC = 2
_NS = 16
_NW = _NC * _NS
_B = 80
_DEPTH = 6
_SEGS = (128000, 128000, 64000)


def _proj_body(nf_ref, nh_ref, w1_ref, b1_ref, pd_ref, ps_ref):
    nf = nf_ref[...]
    nh = nh_ref[...]
    pd_ref[...] = (
        jnp.dot(nf, w1_ref[0:2], preferred_element_type=jnp.float32)
        + jnp.dot(nh, w1_ref[2:130], preferred_element_type=jnp.float32)
        + b1_ref[...]
    )
    ps_ref[...] = (
        jnp.dot(nf, w1_ref[130:132], preferred_element_type=jnp.float32)
        + jnp.dot(nh, w1_ref[132:260], preferred_element_type=jnp.float32)
    )


def _proj(nf, nh, w1, b1_row):
    return pl.pallas_call(
        _proj_body,
        out_shape=(
            jax.ShapeDtypeStruct((_N_NODES, _HID), jnp.float32),
            jax.ShapeDtypeStruct((_N_NODES, _HID), jnp.float32),
        ),
    )(nf, nh, w1, b1_row)


_mesh = plsc.VectorSubcoreMesh(core_axis_name="c", subcore_axis_name="s")


def _make_sc_edge(n_edges, seg_base):
    ept = n_edges // _NW
    nch = ept // _B
    ngrp = nch // _DEPTH
    rem = nch % _DEPTH

    @functools.partial(
        pl.kernel,
        out_type=jax.ShapeDtypeStruct((n_edges, _HID), jnp.float32),
        mesh=_mesh,
        scratch_types=[
            pltpu.VMEM((_DEPTH * _B,), jnp.int32),
            pltpu.VMEM((_DEPTH * _B,), jnp.int32),
            pltpu.VMEM((_DEPTH, _B, _HID), jnp.float32),
            pltpu.VMEM((_DEPTH, _B, _HID), jnp.float32),
            [pltpu.SemaphoreType.DMA] * _DEPTH,
            [pltpu.SemaphoreType.DMA] * _DEPTH,
            pltpu.SemaphoreType.DMA,
        ],
    )
    def sc_edge(pd_hbm, ps_hbm, ei_hbm, h_hbm,
                dsti, srci, gd, gs, semd, sems, semw):
        wid = lax.axis_index("s") * _NC + lax.axis_index("c")
        base0 = wid * ept
        doff = _N_EDGES + seg_base
        soff = seg_base

        def stage_group(c0, n):
            pltpu.sync_copy(ei_hbm.at[pl.ds(doff + base0 + c0 * _B, n * _B)],
                            dsti.at[pl.ds(0, n * _B)])
            pltpu.sync_copy(ei_hbm.at[pl.ds(soff + base0 + c0 * _B, n * _B)],
                            srci.at[pl.ds(0, n * _B)])

        def issue_gathers(j):
            cpd = pltpu.async_copy(pd_hbm.at[dsti.at[pl.ds(j * _B, _B)]],
                                   gd.at[j], semd[j])
            cps = pltpu.async_copy(ps_hbm.at[srci.at[pl.ds(j * _B, _B)]],
                                   gs.at[j], sems[j])
            return cpd, cps

        def compute_chunk(j):
            def row(i, carry):
                for c in range(_HID // 16):
                    sl = pl.ds(c * 16, 16)
                    plsc.addupdate(gd.at[j, i, sl], gs[j, i, sl])
                return carry

            lax.fori_loop(0, _B, row, 0)

        def write_chunk(ci, j):
            return pltpu.async_copy(gd.at[j],
                                    h_hbm.at[pl.ds(base0 + ci * _B, _B)], semw)

        def run_group(c0, n):
            stage_group(c0, n)
            pend = [issue_gathers(j) for j in range(n)]
            writes = []
            for j in range(n):
                cpd, cps = pend[j]
                cpd.wait()
                cps.wait()
                compute_chunk(j)
                writes.append(write_chunk(c0 + j, j))
            for w in writes:
                w.wait()

        def group(g, carry):
            run_group(g * _DEPTH, _DEPTH)
            return carry

        lax.fori_loop(0, ngrp, group, 0)
        if rem:
            run_group(ngrp * _DEPTH, rem)

    return sc_edge


_sc_edges = []
_base = 0
for _n in _SEGS:
    _sc_edges.append(_make_sc_edge(_n, _base))
    _base += _n


def _post_body(h_ref, dist_ref, wd_ref, w2_ref, b2_ref, out_ref):
    be = h_ref.shape[0]
    dr = dist_ref.shape[0]
    d = dist_ref[...]
    p1 = jax.lax.broadcast_in_dim(d, (dr, 128, 128), (0, 2)).reshape(be, 128)
    lane = jax.lax.broadcasted_iota(jnp.int32, (be, 128), 1)
    row = jax.lax.broadcasted_iota(jnp.int32, (be, 128), 0)
    dist_col = jnp.sum(jnp.where(lane == row % 128, p1, 0.0), axis=1,
                       keepdims=True)
    h = jnp.maximum(h_ref[...] + dist_col * wd_ref[...], 0.0)
    out = jnp.dot(h, w2_ref[...], preferred_element_type=jnp.float32) + b2_ref[...]
    out_ref[...] = jnp.maximum(out, 0.0).T


def _post(h, dist2d, wd_row, w2, b2_row, n_edges):
    be = 8192
    grid = pl.cdiv(n_edges, be)
    dr = be // 128
    return pl.pallas_call(
        _post_body,
        grid=(grid,),
        in_specs=[
            pl.BlockSpec((be, _HID), lambda i: (i, 0)),
            pl.BlockSpec((dr, 128), lambda i: (i, 0)),
            pl.BlockSpec((1, _HID), lambda i: (0, 0)),
            pl.BlockSpec((_HID, _EMB), lambda i: (0, 0)),
            pl.BlockSpec((1, _EMB), lambda i: (0, 0)),
        ],
        out_specs=pl.BlockSpec((_EMB, be), lambda i: (0, i)),
        out_shape=jax.ShapeDtypeStruct((_EMB, n_edges), jnp.float32),
    )(h, dist2d, wd_row, w2, b2_row)


def kernel(node_features, node_hidden_state, edge_index, distance, W1, b1, W2, b2):
    pd, ps = _proj(node_features, node_hidden_state, W1, b1.reshape(1, _HID))
    ei1d = edge_index.reshape(2 * _N_EDGES)
    wd_row = W1[260].reshape(1, _HID)
    b2_row = b2.reshape(1, _EMB)
    hs = [sc(pd, ps, ei1d) for sc in _sc_edges]
    outs = []
    base = 0
    for h, n in zip(hs, _SEGS):
        outs.append(_post(h, distance[base:base + n].reshape(n // 128, 128),
                          wd_row, W2, b2_row, n))
        base += n
    return jnp.concatenate(outs, axis=1).T
